# Initial kernel scaffold; baseline (speedup 1.0000x reference)
import jax
import jax.numpy as jnp
from jax.experimental import pallas as pl


def kernel(x, edge_index, edge_attr, batch, problemType, ggc0_weight, ggc0_w_ih, ggc0_w_hh, ggc0_b_ih, ggc0_b_hh, ggc1_weight, ggc1_w_ih, ggc1_w_hh, ggc1_b_ih, ggc1_b_hh, ggc2_weight, ggc2_w_ih, ggc2_w_hh, ggc2_b_ih, ggc2_b_hh, fc1_W, fc1_b, fc2_W, fc2_b, fcLast_W, fcLast_b):
    raise NotImplementedError("write your pallas kernel here")


# speedup vs baseline: 1.3428x; 1.3428x over previous
import functools

import jax
import jax.numpy as jnp
from jax import lax
from jax.experimental import pallas as pl
from jax.experimental.pallas import tpu as pltpu
from jax.experimental.pallas import tpu_sc as plsc

N = 10000
E = 320000
D = 128
T = 3
G = 128
PASSES = 3
R3 = T * N
TAB_R = 30720
QUAR = 8000
NQ = 4
ACC_Q = 8192
LDUMP = QUAR
CHUNK = 128
NCH = 2560
E_PAD = NCH * CHUNK
NKC = NCH // 32
NBUF = 2
BN = 1000
NB = N // BN
QB = QUAR // BN

PACC = 136
PDUMP = G
PCH = 64
NP_PAD = 10240
PNK = NP_PAD // 32 // PCH


@functools.cache
def _mesh():
    return plsc.VectorSubcoreMesh(core_axis_name="c", subcore_axis_name="s")


def _zero_buf(buf, nrow, ncol):
    z = jnp.zeros((16,), jnp.float32)

    def row(r, carry):
        def qcol(q, carry2):
            buf[r, pl.ds(q * 16, 16)] = z
            return carry2
        return lax.fori_loop(0, ncol // 16, qcol, carry)

    lax.fori_loop(0, nrow, row, 0)


def _zero_buf3(buf, nrow, ncol):
    z = jnp.zeros((16,), jnp.float32)

    def row(r, carry):
        def qcol(q, carry2):
            buf[r, 0, pl.ds(q * 16, 16)] = z
            return carry2
        return lax.fori_loop(0, ncol // 16, qcol, carry)

    lax.fori_loop(0, nrow, row, 0)


def _ones_buf(buf, nrow, ncol):
    o = jnp.full((16,), 1.0, jnp.float32)

    def row(r, carry):
        def qcol(q, carry2):
            buf[r, pl.ds(q * 16, 16)] = o
            return carry2
        return lax.fori_loop(0, ncol // 16, qcol, carry)

    lax.fori_loop(0, nrow, row, 0)


def _segsum_body(table, p_hbm, out, g_v, s_v, r0, r1, acc, m0, m1):
    c = lax.axis_index("c")
    s = lax.axis_index("s")
    wid = s * 2 + c
    rows = [r0, r1]
    sems = [m0, m1]
    base = s * (ACC_Q // 16)
    pltpu.sync_copy(p_hbm.at[pl.ds(wid * NKC * CHUNK, NKC * CHUNK)], s_v)
    mask15 = jnp.int32(32767)

    def ug(k, carry):
        iv = s_v[pl.ds(k * 16, 16)]
        g_v[pl.ds(k * 16, 16)] = iv & mask15
        s_v[pl.ds(k * 16, 16)] = iv >> 15
        return carry

    lax.fori_loop(0, NKC * CHUNK // 16, ug, 0)
    _zero_buf(r0, CHUNK, D)

    def zacc(t, carry):
        pltpu.sync_copy(r0, acc.at[pl.ds(base + t * CHUNK, CHUNK)])
        return carry

    lax.fori_loop(0, ACC_Q // 16 // CHUNK, zacc, 0)
    plsc.subcore_barrier()
    for b in range(NBUF):
        pltpu.async_copy(table.at[g_v.at[pl.ds(b * CHUNK, CHUNK)]], rows[b], sems[b])

    def outer(i, carry):
        k0 = i * NBUF
        for b in range(NBUF):
            k = k0 + b
            pltpu.make_async_copy(table.at[g_v.at[pl.ds(k * CHUNK, CHUNK)]], rows[b],
                                  sems[b]).wait()
            pltpu.sync_copy(rows[b], acc.at[s_v.at[pl.ds(k * CHUNK, CHUNK)]], add=True)

            @pl.when(k + NBUF < NKC)
            def _fire(b=b, k=k):
                pltpu.async_copy(table.at[g_v.at[pl.ds((k + NBUF) * CHUNK, CHUNK)]], rows[b],
                                 sems[b])
        return carry

    lax.fori_loop(0, NKC // NBUF, outer, 0)
    plsc.subcore_barrier()

    def dr(t, carry):
        pltpu.sync_copy(acc.at[pl.ds(base + t * CHUNK, CHUNK)], r1)
        pltpu.sync_copy(
            r1, out.at[pl.ds(base + t * CHUNK, CHUNK), pl.ds(c * D, D)])
        return carry

    lax.fori_loop(0, ACC_Q // 16 // CHUNK, dr, 0)


@functools.cache
def _segsum_kernel():
    return pl.kernel(
        _segsum_body,
        out_type=jax.ShapeDtypeStruct((ACC_Q, 2 * D), jnp.float32),
        mesh=_mesh(),
        scratch_types=[
            pltpu.VMEM((NKC * CHUNK,), jnp.int32),
            pltpu.VMEM((NKC * CHUNK,), jnp.int32),
            pltpu.VMEM((CHUNK, D), jnp.float32),
            pltpu.VMEM((CHUNK, D), jnp.float32),
            pltpu.VMEM_SHARED((ACC_Q, D), jnp.float32),
            pltpu.SemaphoreType.DMA,
            pltpu.SemaphoreType.DMA,
        ],
    )


def _segsum_call(table, p_parts):
    parts = []
    for qq in range(NQ):
        o = _segsum_kernel()(table, p_parts[qq])
        parts.append((o[:, :D] + o[:, D:])[:QUAR])
    return jnp.concatenate(parts)[:R3]


def _pool_body(hs, b_hbm, out_s, out_c, b_v, hbuf, ones_v, zb, cb, acc, accc):
    c = lax.axis_index("c")
    s = lax.axis_index("s")
    wid = s * 2 + c
    _zero_buf(zb, PACC, D)
    _zero_buf(cb, PACC, D)
    _ones_buf(ones_v, PCH, D)

    @pl.when(s == 0)
    def _z():
        pltpu.sync_copy(zb, acc)
        pltpu.sync_copy(cb, accc)

    pltpu.sync_copy(b_hbm.at[pl.ds(wid * PNK * PCH, PNK * PCH)], b_v)
    plsc.subcore_barrier()

    def lp(k, carry):
        pltpu.sync_copy(hs.at[pl.ds(wid * (PNK * PCH) + k * PCH, PCH)], hbuf)
        pltpu.sync_copy(hbuf, acc.at[b_v.at[pl.ds(k * PCH, PCH)]], add=True)
        pltpu.sync_copy(ones_v, accc.at[b_v.at[pl.ds(k * PCH, PCH)]], add=True)
        return carry

    lax.fori_loop(0, PNK, lp, 0)
    plsc.subcore_barrier()

    @pl.when(s == 0)
    def _d():
        pltpu.sync_copy(acc, zb)
        pltpu.sync_copy(zb, out_s.at[c])
        pltpu.sync_copy(accc, cb)
        pltpu.sync_copy(cb, out_c.at[c])


@functools.cache
def _pool_kernel():
    return pl.kernel(
        _pool_body,
        out_type=(
            jax.ShapeDtypeStruct((2, PACC, D), jnp.float32),
            jax.ShapeDtypeStruct((2, PACC, D), jnp.float32),
        ),
        mesh=_mesh(),
        scratch_types=[
            pltpu.VMEM((PNK * PCH,), jnp.int32),
            pltpu.VMEM((PCH, D), jnp.float32),
            pltpu.VMEM((PCH, D), jnp.float32),
            pltpu.VMEM((PACC, D), jnp.float32),
            pltpu.VMEM((PACC, D), jnp.float32),
            pltpu.VMEM_SHARED((PACC, D), jnp.float32),
            pltpu.VMEM_SHARED((PACC, D), jnp.float32),
        ],
    )


def _pool_call(h, bpad):
    return _pool_kernel()(h, bpad)


def _mm_body(h_ref, w_ref, o_ref):
    o_ref[...] = jnp.dot(h_ref[0], w_ref[0],
                         preferred_element_type=jnp.float32)


def _mm_call(h_all, w):
    return pl.pallas_call(
        _mm_body,
        grid=(T, NB),
        in_specs=[
            pl.BlockSpec((1, BN, D), lambda j, i: (j, i, 0)),
            pl.BlockSpec((1, D, D), lambda j, i: (j, 0, 0)),
        ],
        out_specs=pl.BlockSpec((BN, D), lambda j, i: (j * NB + i, 0)),
        out_shape=jax.ShapeDtypeStruct((TAB_R, D), jnp.float32),
    )(h_all, w)


def _gru_body(s_ref, inv_ref, h_ref, wi_ref, wh_ref, bi_ref, bh_ref, o_ref):
    agg = s_ref[...] * inv_ref[...]
    hb = h_ref[0]
    gi = jnp.dot(agg, wi_ref[0], preferred_element_type=jnp.float32) + bi_ref[0]
    gh = jnp.dot(hb, wh_ref[0], preferred_element_type=jnp.float32) + bh_ref[0]
    r = jax.nn.sigmoid(gi[:, :D] + gh[:, :D])
    z = jax.nn.sigmoid(gi[:, D:2 * D] + gh[:, D:2 * D])
    n = jnp.tanh(gi[:, 2 * D:] + r * gh[:, 2 * D:])
    o_ref[0] = (1.0 - z) * n + z * hb


def _gru_call(sums, inv_in, h_all, wiT, whT, bi, bh):
    return pl.pallas_call(
        _gru_body,
        grid=(T, NB),
        in_specs=[
            pl.BlockSpec((BN, D), lambda j, i: (j * NB + i, 0)),
            pl.BlockSpec((BN, 1), lambda j, i: (j * NB + i, 0)),
            pl.BlockSpec((1, BN, D), lambda j, i: (j, i, 0)),
            pl.BlockSpec((1, D, 3 * D), lambda j, i: (j, 0, 0)),
            pl.BlockSpec((1, D, 3 * D), lambda j, i: (j, 0, 0)),
            pl.BlockSpec((1, 1, 3 * D), lambda j, i: (j, 0, 0)),
            pl.BlockSpec((1, 1, 3 * D), lambda j, i: (j, 0, 0)),
        ],
        out_specs=pl.BlockSpec((1, BN, D), lambda j, i: (j, i, 0)),
        out_shape=jax.ShapeDtypeStruct((T, N, D), jnp.float32),
    )(sums, inv_in, h_all, wiT, whT, bi, bh)


def _epi_body(h_ref, c_ref, o_ref):
    hb = h_ref[...]
    cc = c_ref[...]
    comb = hb[0] * cc[0] + hb[1] * cc[1] + hb[2] * cc[2]
    o_ref[...] = jnp.broadcast_to(comb[None], (T, BN, D))


def _epi_call(h_all, coef_in):
    return pl.pallas_call(
        _epi_body,
        grid=(NB,),
        in_specs=[
            pl.BlockSpec((T, BN, D), lambda i: (0, i, 0)),
            pl.BlockSpec((T, 1, D), lambda i: (0, 0, 0)),
        ],
        out_specs=pl.BlockSpec((T, BN, D), lambda i: (0, i, 0)),
        out_shape=jax.ShapeDtypeStruct((T, N, D), jnp.float32),
    )(h_all, coef_in)


def _epf_body(h_ref, c_ref, o_ref):
    hb = h_ref[...]
    cc = c_ref[...]
    o_ref[...] = hb[0] * cc[0] + hb[1] * cc[1] + hb[2] * cc[2]


def _epf_call(h_all, coef_in):
    return pl.pallas_call(
        _epf_body,
        grid=(NB,),
        in_specs=[
            pl.BlockSpec((T, BN, D), lambda i: (0, i, 0)),
            pl.BlockSpec((T, 1, D), lambda i: (0, 0, 0)),
        ],
        out_specs=pl.BlockSpec((BN, D), lambda i: (i, 0)),
        out_shape=jax.ShapeDtypeStruct((NP_PAD, D), jnp.float32),
    )(h_all, coef_in)


def _fc_body(ps_ref, pc_ref, pt_ref, w1a_ref, w1b_ref, b1_ref, w2_ref,
             b2_ref, wl_ref, bl_ref, o_ref):
    sums = (ps_ref[0] + ps_ref[1])[:G]
    cnt = (pc_ref[0] + pc_ref[1])[:G, 0:1]
    pooled = sums / jnp.maximum(cnt, 1.0)
    x = (jnp.dot(pooled, w1a_ref[...], preferred_element_type=jnp.float32)
         + jnp.dot(pt_ref[...], w1b_ref[...], preferred_element_type=jnp.float32)
         + b1_ref[...])
    x = jnp.where(x > 0, x, 0.01 * x)
    x = jnp.dot(x, w2_ref[...], preferred_element_type=jnp.float32) + b2_ref[...]
    x = jnp.where(x > 0, x, 0.01 * x)
    o_ref[...] = jnp.dot(x, wl_ref[...], preferred_element_type=jnp.float32) + bl_ref[...]


def _fc_call(ps, pc, pt, w1a, w1b, b1, w2, b2, wl, bl):
    return pl.pallas_call(
        _fc_body,
        out_shape=jax.ShapeDtypeStruct((G, 2), jnp.float32),
    )(ps, pc, pt, w1a, w1b, b1, w2, b2, wl, bl)


def kernel(x, edge_index, edge_attr, batch, problemType,
           ggc0_weight, ggc0_w_ih, ggc0_w_hh, ggc0_b_ih, ggc0_b_hh,
           ggc1_weight, ggc1_w_ih, ggc1_w_hh, ggc1_b_ih, ggc1_b_hh,
           ggc2_weight, ggc2_w_ih, ggc2_w_hh, ggc2_b_ih, ggc2_b_hh,
           fc1_W, fc1_b, fc2_W, fc2_b, fcLast_W, fcLast_b):
    f32 = jnp.float32
    src = edge_index[0].astype(jnp.int32)
    dst = edge_index[1].astype(jnp.int32)
    ea = edge_attr.astype(jnp.int32)
    onehot = ea[None, :] == jnp.arange(T, dtype=jnp.int32)[:, None]
    present = jnp.any(onehot, axis=1)
    rank = jnp.cumsum(present.astype(jnp.int32)) - 1
    num_vals = jnp.sum(present.astype(f32))
    slot = jnp.take(rank, ea)
    g = slot * N + src
    sdx = slot * N + dst
    gflat = jnp.concatenate([g, jnp.zeros((E_PAD - E,), jnp.int32)])
    sflat = jnp.concatenate([sdx, jnp.full((E_PAD - E,), R3, jnp.int32)])
    p_parts = []
    for qq in range(NQ):
        lo = qq * QUAR
        sq = jnp.where((sflat >= lo) & (sflat < lo + QUAR), sflat - lo, LDUMP)
        p_parts.append(gflat | (sq << 15))
    coef = (num_vals > jnp.arange(T, dtype=f32)).astype(f32) / num_vals
    coef_in = jnp.broadcast_to(coef[:, None, None], (T, 1, D))

    ones_tbl = jnp.ones((TAB_R, D), f32)
    cnt2 = _segsum_call(ones_tbl, p_parts)
    inv_in = 1.0 / jnp.maximum(cnt2[:, 0:1], 1.0)

    w_stack = jnp.stack([ggc0_weight, ggc1_weight, ggc2_weight])
    wiT = jnp.stack([ggc0_w_ih.T, ggc1_w_ih.T, ggc2_w_ih.T])
    whT = jnp.stack([ggc0_w_hh.T, ggc1_w_hh.T, ggc2_w_hh.T])
    bi = jnp.stack([ggc0_b_ih, ggc1_b_ih, ggc2_b_ih])[:, None, :]
    bh = jnp.stack([ggc0_b_hh, ggc1_b_hh, ggc2_b_hh])[:, None, :]

    h_all = jnp.broadcast_to(x[None], (T, N, D))
    for p in range(PASSES):
        for l in range(2):
            tbl = _mm_call(h_all, w_stack[:, l])
            sums = _segsum_call(tbl, p_parts)
            h_all = _gru_call(sums, inv_in, h_all, wiT, whT, bi, bh)
        if p + 1 < PASSES:
            h_all = _epi_call(h_all, coef_in)
    h_fin = _epf_call(h_all, coef_in)

    bpad = jnp.concatenate(
        [batch.astype(jnp.int32), jnp.full((NP_PAD - N,), PDUMP, jnp.int32)])
    ps, pc = _pool_call(h_fin, bpad)

    return _fc_call(
        ps, pc, problemType,
        fc1_W[:, :D].T, fc1_W[:, D:].T, fc1_b[None],
        fc2_W.T, fc2_b[None], fcLast_W.T, fcLast_b[None],
    )


# speedup vs baseline: 1.9897x; 1.4817x over previous
import functools

import jax
import jax.numpy as jnp
from jax import lax
from jax.experimental import pallas as pl
from jax.experimental.pallas import tpu as pltpu
from jax.experimental.pallas import tpu_sc as plsc

N = 10000
E = 320000
D = 128
T = 3
G = 128
PASSES = 3
R3 = T * N
TAB_R = 30720
QUAR = 10000
NQ = 3
ACC_Q = 10240
WCH = 64
NKW = 160
LDUMP = QUAR
CHUNK = 128
NCH = 2560
E_PAD = NCH * CHUNK
NKC = NCH // 32
NBUF = 2
BN = 1000
NB = N // BN
QB = QUAR // BN

PACC = 136
PDUMP = G
PCH = 64
NP_PAD = 10240
PNK = NP_PAD // 32 // PCH


@functools.cache
def _mesh():
    return plsc.VectorSubcoreMesh(core_axis_name="c", subcore_axis_name="s")


def _zero_buf(buf, nrow, ncol):
    z = jnp.zeros((16,), jnp.float32)

    def row(r, carry):
        def qcol(q, carry2):
            buf[r, pl.ds(q * 16, 16)] = z
            return carry2
        return lax.fori_loop(0, ncol // 16, qcol, carry)

    lax.fori_loop(0, nrow, row, 0)


def _zero_buf3(buf, nrow, ncol):
    z = jnp.zeros((16,), jnp.float32)

    def row(r, carry):
        def qcol(q, carry2):
            buf[r, 0, pl.ds(q * 16, 16)] = z
            return carry2
        return lax.fori_loop(0, ncol // 16, qcol, carry)

    lax.fori_loop(0, nrow, row, 0)


def _ones_buf(buf, nrow, ncol):
    o = jnp.full((16,), 1.0, jnp.float32)

    def row(r, carry):
        def qcol(q, carry2):
            buf[r, pl.ds(q * 16, 16)] = o
            return carry2
        return lax.fori_loop(0, ncol // 16, qcol, carry)

    lax.fori_loop(0, nrow, row, 0)


def _segsum_body(table, p_hbm, out, g_v, s_v, r0, r1, acc, m0, m1):
    c = lax.axis_index("c")
    s = lax.axis_index("s")
    wid = s * 2 + c
    rows = [r0, r1]
    sems = [m0, m1]
    base = s * (ACC_Q // 16)
    pltpu.sync_copy(p_hbm.at[pl.ds(wid * NKC * CHUNK, NKC * CHUNK)], s_v)
    mask15 = jnp.int32(32767)

    def ug(k, carry):
        iv = s_v[pl.ds(k * 16, 16)]
        g_v[pl.ds(k * 16, 16)] = iv & mask15
        s_v[pl.ds(k * 16, 16)] = iv >> 15
        return carry

    lax.fori_loop(0, NKC * CHUNK // 16, ug, 0)
    _zero_buf(r0, WCH, D)

    def zacc(t, carry):
        pltpu.sync_copy(r0, acc.at[pl.ds(base + t * WCH, WCH)])
        return carry

    lax.fori_loop(0, ACC_Q // 16 // WCH, zacc, 0)
    plsc.subcore_barrier()
    for b in range(NBUF):
        pltpu.async_copy(table.at[g_v.at[pl.ds(b * WCH, WCH)]], rows[b], sems[b])

    def outer(i, carry):
        k0 = i * NBUF
        for b in range(NBUF):
            k = k0 + b
            pltpu.make_async_copy(table.at[g_v.at[pl.ds(k * WCH, WCH)]], rows[b],
                                  sems[b]).wait()
            pltpu.sync_copy(rows[b], acc.at[s_v.at[pl.ds(k * WCH, WCH)]], add=True)

            @pl.when(k + NBUF < NKW)
            def _fire(b=b, k=k):
                pltpu.async_copy(table.at[g_v.at[pl.ds((k + NBUF) * WCH, WCH)]], rows[b],
                                 sems[b])
        return carry

    lax.fori_loop(0, NKW // NBUF, outer, 0)
    plsc.subcore_barrier()

    def dr(t, carry):
        pltpu.sync_copy(acc.at[pl.ds(base + t * WCH, WCH)], r1)
        pltpu.sync_copy(
            r1, out.at[pl.ds(base + t * WCH, WCH), pl.ds(c * D, D)])
        return carry

    lax.fori_loop(0, ACC_Q // 16 // WCH, dr, 0)


@functools.cache
def _segsum_kernel():
    return pl.kernel(
        _segsum_body,
        out_type=jax.ShapeDtypeStruct((ACC_Q, 2 * D), jnp.float32),
        mesh=_mesh(),
        scratch_types=[
            pltpu.VMEM((NKC * CHUNK,), jnp.int32),
            pltpu.VMEM((NKC * CHUNK,), jnp.int32),
            pltpu.VMEM((WCH, D), jnp.float32),
            pltpu.VMEM((WCH, D), jnp.float32),
            pltpu.VMEM_SHARED((ACC_Q, D), jnp.float32),
            pltpu.SemaphoreType.DMA,
            pltpu.SemaphoreType.DMA,
        ],
    )


def _segsum_call(table, p_parts):
    parts = []
    for qq in range(NQ):
        o = _segsum_kernel()(table, p_parts[qq])
        parts.append((o[:, :D] + o[:, D:])[:QUAR])
    return jnp.concatenate(parts)


def _pool_body(hs, b_hbm, out_s, out_c, b_v, hbuf, ones_v, zb, cb, acc, accc):
    c = lax.axis_index("c")
    s = lax.axis_index("s")
    wid = s * 2 + c
    _zero_buf(zb, PACC, D)
    _zero_buf(cb, PACC, D)
    _ones_buf(ones_v, PCH, D)

    @pl.when(s == 0)
    def _z():
        pltpu.sync_copy(zb, acc)
        pltpu.sync_copy(cb, accc)

    pltpu.sync_copy(b_hbm.at[pl.ds(wid * PNK * PCH, PNK * PCH)], b_v)
    plsc.subcore_barrier()

    def lp(k, carry):
        pltpu.sync_copy(hs.at[pl.ds(wid * (PNK * PCH) + k * PCH, PCH)], hbuf)
        pltpu.sync_copy(hbuf, acc.at[b_v.at[pl.ds(k * PCH, PCH)]], add=True)
        pltpu.sync_copy(ones_v, accc.at[b_v.at[pl.ds(k * PCH, PCH)]], add=True)
        return carry

    lax.fori_loop(0, PNK, lp, 0)
    plsc.subcore_barrier()

    @pl.when(s == 0)
    def _d():
        pltpu.sync_copy(acc, zb)
        pltpu.sync_copy(zb, out_s.at[c])
        pltpu.sync_copy(accc, cb)
        pltpu.sync_copy(cb, out_c.at[c])


@functools.cache
def _pool_kernel():
    return pl.kernel(
        _pool_body,
        out_type=(
            jax.ShapeDtypeStruct((2, PACC, D), jnp.float32),
            jax.ShapeDtypeStruct((2, PACC, D), jnp.float32),
        ),
        mesh=_mesh(),
        scratch_types=[
            pltpu.VMEM((PNK * PCH,), jnp.int32),
            pltpu.VMEM((PCH, D), jnp.float32),
            pltpu.VMEM((PCH, D), jnp.float32),
            pltpu.VMEM((PACC, D), jnp.float32),
            pltpu.VMEM((PACC, D), jnp.float32),
            pltpu.VMEM_SHARED((PACC, D), jnp.float32),
            pltpu.VMEM_SHARED((PACC, D), jnp.float32),
        ],
    )


def _pool_call(h, bpad):
    return _pool_kernel()(h, bpad)


def _mm_body(h_ref, w_ref, o_ref):
    o_ref[...] = jnp.dot(h_ref[0], w_ref[0],
                         preferred_element_type=jnp.float32)


def _mm_call(h_all, w):
    return pl.pallas_call(
        _mm_body,
        grid=(T, NB),
        in_specs=[
            pl.BlockSpec((1, BN, D), lambda j, i: (j, i, 0)),
            pl.BlockSpec((1, D, D), lambda j, i: (j, 0, 0)),
        ],
        out_specs=pl.BlockSpec((BN, D), lambda j, i: (j * NB + i, 0)),
        out_shape=jax.ShapeDtypeStruct((TAB_R, D), jnp.float32),
    )(h_all, w)


def _gru_body(s_ref, inv_ref, h_ref, wi_ref, wh_ref, bi_ref, bh_ref, o_ref):
    agg = s_ref[...] * inv_ref[...]
    hb = h_ref[0]
    gi = jnp.dot(agg, wi_ref[0], preferred_element_type=jnp.float32) + bi_ref[0]
    gh = jnp.dot(hb, wh_ref[0], preferred_element_type=jnp.float32) + bh_ref[0]
    r = jax.nn.sigmoid(gi[:, :D] + gh[:, :D])
    z = jax.nn.sigmoid(gi[:, D:2 * D] + gh[:, D:2 * D])
    n = jnp.tanh(gi[:, 2 * D:] + r * gh[:, 2 * D:])
    o_ref[0] = (1.0 - z) * n + z * hb


def _gru_call(sums, inv_in, h_all, wiT, whT, bi, bh):
    return pl.pallas_call(
        _gru_body,
        grid=(T, NB),
        in_specs=[
            pl.BlockSpec((BN, D), lambda j, i: (j * NB + i, 0)),
            pl.BlockSpec((BN, 1), lambda j, i: (j * NB + i, 0)),
            pl.BlockSpec((1, BN, D), lambda j, i: (j, i, 0)),
            pl.BlockSpec((1, D, 3 * D), lambda j, i: (j, 0, 0)),
            pl.BlockSpec((1, D, 3 * D), lambda j, i: (j, 0, 0)),
            pl.BlockSpec((1, 1, 3 * D), lambda j, i: (j, 0, 0)),
            pl.BlockSpec((1, 1, 3 * D), lambda j, i: (j, 0, 0)),
        ],
        out_specs=pl.BlockSpec((1, BN, D), lambda j, i: (j, i, 0)),
        out_shape=jax.ShapeDtypeStruct((T, N, D), jnp.float32),
    )(sums, inv_in, h_all, wiT, whT, bi, bh)


def _epi_body(h_ref, c_ref, o_ref):
    hb = h_ref[...]
    cc = c_ref[...]
    comb = hb[0] * cc[0] + hb[1] * cc[1] + hb[2] * cc[2]
    o_ref[...] = jnp.broadcast_to(comb[None], (T, BN, D))


def _epi_call(h_all, coef_in):
    return pl.pallas_call(
        _epi_body,
        grid=(NB,),
        in_specs=[
            pl.BlockSpec((T, BN, D), lambda i: (0, i, 0)),
            pl.BlockSpec((T, 1, D), lambda i: (0, 0, 0)),
        ],
        out_specs=pl.BlockSpec((T, BN, D), lambda i: (0, i, 0)),
        out_shape=jax.ShapeDtypeStruct((T, N, D), jnp.float32),
    )(h_all, coef_in)


def _epf_body(h_ref, c_ref, o_ref):
    hb = h_ref[...]
    cc = c_ref[...]
    o_ref[...] = hb[0] * cc[0] + hb[1] * cc[1] + hb[2] * cc[2]


def _epf_call(h_all, coef_in):
    return pl.pallas_call(
        _epf_body,
        grid=(NB,),
        in_specs=[
            pl.BlockSpec((T, BN, D), lambda i: (0, i, 0)),
            pl.BlockSpec((T, 1, D), lambda i: (0, 0, 0)),
        ],
        out_specs=pl.BlockSpec((BN, D), lambda i: (i, 0)),
        out_shape=jax.ShapeDtypeStruct((NP_PAD, D), jnp.float32),
    )(h_all, coef_in)


def _fc_body(ps_ref, pc_ref, pt_ref, w1a_ref, w1b_ref, b1_ref, w2_ref,
             b2_ref, wl_ref, bl_ref, o_ref):
    sums = (ps_ref[0] + ps_ref[1])[:G]
    cnt = (pc_ref[0] + pc_ref[1])[:G, 0:1]
    pooled = sums / jnp.maximum(cnt, 1.0)
    x = (jnp.dot(pooled, w1a_ref[...], preferred_element_type=jnp.float32)
         + jnp.dot(pt_ref[...], w1b_ref[...], preferred_element_type=jnp.float32)
         + b1_ref[...])
    x = jnp.where(x > 0, x, 0.01 * x)
    x = jnp.dot(x, w2_ref[...], preferred_element_type=jnp.float32) + b2_ref[...]
    x = jnp.where(x > 0, x, 0.01 * x)
    o_ref[...] = jnp.dot(x, wl_ref[...], preferred_element_type=jnp.float32) + bl_ref[...]


def _fc_call(ps, pc, pt, w1a, w1b, b1, w2, b2, wl, bl):
    return pl.pallas_call(
        _fc_body,
        out_shape=jax.ShapeDtypeStruct((G, 2), jnp.float32),
    )(ps, pc, pt, w1a, w1b, b1, w2, b2, wl, bl)


def kernel(x, edge_index, edge_attr, batch, problemType,
           ggc0_weight, ggc0_w_ih, ggc0_w_hh, ggc0_b_ih, ggc0_b_hh,
           ggc1_weight, ggc1_w_ih, ggc1_w_hh, ggc1_b_ih, ggc1_b_hh,
           ggc2_weight, ggc2_w_ih, ggc2_w_hh, ggc2_b_ih, ggc2_b_hh,
           fc1_W, fc1_b, fc2_W, fc2_b, fcLast_W, fcLast_b):
    f32 = jnp.float32
    src = edge_index[0].astype(jnp.int32)
    dst = edge_index[1].astype(jnp.int32)
    ea = edge_attr.astype(jnp.int32)
    onehot = ea[None, :] == jnp.arange(T, dtype=jnp.int32)[:, None]
    present = jnp.any(onehot, axis=1)
    rank = jnp.cumsum(present.astype(jnp.int32)) - 1
    num_vals = jnp.sum(present.astype(f32))
    slot = jnp.take(rank, ea)
    g = slot * N + src
    sdx = slot * N + dst
    gflat = jnp.concatenate([g, jnp.zeros((E_PAD - E,), jnp.int32)])
    sflat = jnp.concatenate([sdx, jnp.full((E_PAD - E,), R3, jnp.int32)])
    p_parts = []
    for qq in range(NQ):
        lo = qq * QUAR
        sq = jnp.where((sflat >= lo) & (sflat < lo + QUAR), sflat - lo, LDUMP)
        p_parts.append(gflat | (sq << 15))
    coef = (num_vals > jnp.arange(T, dtype=f32)).astype(f32) / num_vals
    coef_in = jnp.broadcast_to(coef[:, None, None], (T, 1, D))

    ones_tbl = jnp.ones((TAB_R, D), f32)
    cnt2 = _segsum_call(ones_tbl, p_parts)
    inv_in = 1.0 / jnp.maximum(cnt2[:, 0:1], 1.0)

    w_stack = jnp.stack([ggc0_weight, ggc1_weight, ggc2_weight])
    wiT = jnp.stack([ggc0_w_ih.T, ggc1_w_ih.T, ggc2_w_ih.T])
    whT = jnp.stack([ggc0_w_hh.T, ggc1_w_hh.T, ggc2_w_hh.T])
    bi = jnp.stack([ggc0_b_ih, ggc1_b_ih, ggc2_b_ih])[:, None, :]
    bh = jnp.stack([ggc0_b_hh, ggc1_b_hh, ggc2_b_hh])[:, None, :]

    h_all = jnp.broadcast_to(x[None], (T, N, D))
    for p in range(PASSES):
        for l in range(2):
            tbl = _mm_call(h_all, w_stack[:, l])
            sums = _segsum_call(tbl, p_parts)
            h_all = _gru_call(sums, inv_in, h_all, wiT, whT, bi, bh)
        if p + 1 < PASSES:
            h_all = _epi_call(h_all, coef_in)
    h_fin = _epf_call(h_all, coef_in)

    bpad = jnp.concatenate(
        [batch.astype(jnp.int32), jnp.full((NP_PAD - N,), PDUMP, jnp.int32)])
    ps, pc = _pool_call(h_fin, bpad)

    return _fc_call(
        ps, pc, problemType,
        fc1_W[:, :D].T, fc1_W[:, D:].T, fc1_b[None],
        fc2_W.T, fc2_b[None], fcLast_W.T, fcLast_b[None],
    )


# speedup vs baseline: 2.2115x; 1.1115x over previous
import functools

import jax
import jax.numpy as jnp
from jax import lax
from jax.experimental import pallas as pl
from jax.experimental.pallas import tpu as pltpu
from jax.experimental.pallas import tpu_sc as plsc

N = 10000
E = 320000
D = 128
T = 3
G = 128
PASSES = 3
R3 = T * N
TAB_R = 30720
QUAR = 10000
NQ = 3
ACC_Q = 10240
WCH = 64
NKW = 160
LDUMP = QUAR
CHUNK = 128
NCH = 2560
E_PAD = NCH * CHUNK
NKC = NCH // 32
NBUF = 3
BN = 1000
NB = N // BN
QB = QUAR // BN

PACC = 136
PDUMP = G
PCH = 64
NP_PAD = 10240
PNK = NP_PAD // 32 // PCH


@functools.cache
def _mesh():
    return plsc.VectorSubcoreMesh(core_axis_name="c", subcore_axis_name="s")


def _zero_buf(buf, nrow, ncol):
    z = jnp.zeros((16,), jnp.float32)

    def row(r, carry):
        def qcol(q, carry2):
            buf[r, pl.ds(q * 16, 16)] = z
            return carry2
        return lax.fori_loop(0, ncol // 16, qcol, carry)

    lax.fori_loop(0, nrow, row, 0)


def _zero_buf3(buf, nrow, ncol):
    z = jnp.zeros((16,), jnp.float32)

    def row(r, carry):
        def qcol(q, carry2):
            buf[r, 0, pl.ds(q * 16, 16)] = z
            return carry2
        return lax.fori_loop(0, ncol // 16, qcol, carry)

    lax.fori_loop(0, nrow, row, 0)


def _ones_buf(buf, nrow, ncol):
    o = jnp.full((16,), 1.0, jnp.float32)

    def row(r, carry):
        def qcol(q, carry2):
            buf[r, pl.ds(q * 16, 16)] = o
            return carry2
        return lax.fori_loop(0, ncol // 16, qcol, carry)

    lax.fori_loop(0, nrow, row, 0)


def _segsum_body(table, p_hbm, out, g_v, s_v, r0, r1, r2, acc, m0, m1, m2):
    c = lax.axis_index("c")
    s = lax.axis_index("s")
    wid = s * 2 + c
    rows = [r0, r1, r2]
    sems = [m0, m1, m2]
    base = s * (ACC_Q // 16)
    pltpu.sync_copy(p_hbm.at[pl.ds(wid * NKC * CHUNK, NKC * CHUNK)], s_v)
    mask15 = jnp.int32(32767)

    def ug(k, carry):
        iv = s_v[pl.ds(k * 16, 16)]
        g_v[pl.ds(k * 16, 16)] = iv & mask15
        s_v[pl.ds(k * 16, 16)] = iv >> 15
        return carry

    lax.fori_loop(0, NKC * CHUNK // 16, ug, 0)
    _zero_buf(r0, WCH, D)

    def zacc(t, carry):
        pltpu.sync_copy(r0, acc.at[pl.ds(base + t * WCH, WCH)])
        return carry

    lax.fori_loop(0, ACC_Q // 16 // WCH, zacc, 0)
    plsc.subcore_barrier()
    for b in range(NBUF):
        pltpu.async_copy(table.at[g_v.at[pl.ds(b * WCH, WCH)]], rows[b], sems[b])

    def outer(i, carry):
        k0 = i * NBUF
        for b in range(NBUF):
            k = k0 + b
            pltpu.make_async_copy(table.at[g_v.at[pl.ds(k * WCH, WCH)]], rows[b],
                                  sems[b]).wait()
            pltpu.sync_copy(rows[b], acc.at[s_v.at[pl.ds(k * WCH, WCH)]], add=True)

            @pl.when(k + NBUF < NKW)
            def _fire(b=b, k=k):
                pltpu.async_copy(table.at[g_v.at[pl.ds((k + NBUF) * WCH, WCH)]], rows[b],
                                 sems[b])
        return carry

    lax.fori_loop(0, NKW // NBUF, outer, 0)
    for k in range(NKW - NKW % NBUF, NKW):
        b = (NKW - NKW % NBUF) % NBUF
        pltpu.make_async_copy(table.at[g_v.at[pl.ds(k * WCH, WCH)]], rows[b],
                              sems[b]).wait()
        pltpu.sync_copy(rows[b], acc.at[s_v.at[pl.ds(k * WCH, WCH)]], add=True)
    plsc.subcore_barrier()

    def dr(t, carry):
        pltpu.sync_copy(acc.at[pl.ds(base + t * WCH, WCH)], r1)
        pltpu.sync_copy(
            r1, out.at[pl.ds(base + t * WCH, WCH), pl.ds(c * D, D)])
        return carry

    lax.fori_loop(0, ACC_Q // 16 // WCH, dr, 0)


@functools.cache
def _segsum_kernel():
    return pl.kernel(
        _segsum_body,
        out_type=jax.ShapeDtypeStruct((ACC_Q, 2 * D), jnp.float32),
        mesh=_mesh(),
        scratch_types=[
            pltpu.VMEM((NKC * CHUNK,), jnp.int32),
            pltpu.VMEM((NKC * CHUNK,), jnp.int32),
            pltpu.VMEM((WCH, D), jnp.float32),
            pltpu.VMEM((WCH, D), jnp.float32),
            pltpu.VMEM((WCH, D), jnp.float32),
            pltpu.VMEM_SHARED((ACC_Q, D), jnp.float32),
            pltpu.SemaphoreType.DMA,
            pltpu.SemaphoreType.DMA,
            pltpu.SemaphoreType.DMA,
        ],
    )


def _segsum_call(table, p_parts):
    parts = []
    for qq in range(NQ):
        o = _segsum_kernel()(table, p_parts[qq])
        parts.append((o[:, :D] + o[:, D:])[:QUAR])
    return jnp.concatenate(parts)


def _counts_body(p_hbm, out, s_v, ones_v, zb, acc):
    c = lax.axis_index("c")
    s = lax.axis_index("s")
    wid = s * 2 + c
    base = s * (ACC_Q // 16)
    pltpu.sync_copy(p_hbm.at[pl.ds(wid * NKC * CHUNK, NKC * CHUNK)], s_v)

    def ug(k, carry):
        s_v[pl.ds(k * 16, 16)] = s_v[pl.ds(k * 16, 16)] >> 15
        return carry

    lax.fori_loop(0, NKC * CHUNK // 16, ug, 0)
    _zero_buf(zb, WCH, D)
    _ones_buf(ones_v, WCH, D)

    def zacc(t, carry):
        pltpu.sync_copy(zb, acc.at[pl.ds(base + t * WCH, WCH)])
        return carry

    lax.fori_loop(0, ACC_Q // 16 // WCH, zacc, 0)
    plsc.subcore_barrier()

    def lp(k, carry):
        pltpu.sync_copy(ones_v, acc.at[s_v.at[pl.ds(k * WCH, WCH)]], add=True)
        return carry

    lax.fori_loop(0, NKW, lp, 0)
    plsc.subcore_barrier()

    def dr(t, carry):
        pltpu.sync_copy(acc.at[pl.ds(base + t * WCH, WCH)], zb)
        pltpu.sync_copy(
            zb, out.at[pl.ds(base + t * WCH, WCH), pl.ds(c * D, D)])
        return carry

    lax.fori_loop(0, ACC_Q // 16 // WCH, dr, 0)


@functools.cache
def _counts_kernel():
    return pl.kernel(
        _counts_body,
        out_type=jax.ShapeDtypeStruct((ACC_Q, 2 * D), jnp.float32),
        mesh=_mesh(),
        scratch_types=[
            pltpu.VMEM((NKC * CHUNK,), jnp.int32),
            pltpu.VMEM((WCH, D), jnp.float32),
            pltpu.VMEM((WCH, D), jnp.float32),
            pltpu.VMEM_SHARED((ACC_Q, D), jnp.float32),
        ],
    )


def _counts_call(p_parts):
    parts = []
    for qq in range(NQ):
        o = _counts_kernel()(p_parts[qq])
        parts.append((o[:, 0:1] + o[:, D:D + 1])[:QUAR])
    return jnp.concatenate(parts)


def _pool_body(hs, b_hbm, out_s, out_c, b_v, hbuf, ones_v, zb, cb, acc, accc):
    c = lax.axis_index("c")
    s = lax.axis_index("s")
    wid = s * 2 + c
    _zero_buf(zb, PACC, D)
    _zero_buf(cb, PACC, D)
    _ones_buf(ones_v, PCH, D)

    @pl.when(s == 0)
    def _z():
        pltpu.sync_copy(zb, acc)
        pltpu.sync_copy(cb, accc)

    pltpu.sync_copy(b_hbm.at[pl.ds(wid * PNK * PCH, PNK * PCH)], b_v)
    plsc.subcore_barrier()

    def lp(k, carry):
        pltpu.sync_copy(hs.at[pl.ds(wid * (PNK * PCH) + k * PCH, PCH)], hbuf)
        pltpu.sync_copy(hbuf, acc.at[b_v.at[pl.ds(k * PCH, PCH)]], add=True)
        pltpu.sync_copy(ones_v, accc.at[b_v.at[pl.ds(k * PCH, PCH)]], add=True)
        return carry

    lax.fori_loop(0, PNK, lp, 0)
    plsc.subcore_barrier()

    @pl.when(s == 0)
    def _d():
        pltpu.sync_copy(acc, zb)
        pltpu.sync_copy(zb, out_s.at[c])
        pltpu.sync_copy(accc, cb)
        pltpu.sync_copy(cb, out_c.at[c])


@functools.cache
def _pool_kernel():
    return pl.kernel(
        _pool_body,
        out_type=(
            jax.ShapeDtypeStruct((2, PACC, D), jnp.float32),
            jax.ShapeDtypeStruct((2, PACC, D), jnp.float32),
        ),
        mesh=_mesh(),
        scratch_types=[
            pltpu.VMEM((PNK * PCH,), jnp.int32),
            pltpu.VMEM((PCH, D), jnp.float32),
            pltpu.VMEM((PCH, D), jnp.float32),
            pltpu.VMEM((PACC, D), jnp.float32),
            pltpu.VMEM((PACC, D), jnp.float32),
            pltpu.VMEM_SHARED((PACC, D), jnp.float32),
            pltpu.VMEM_SHARED((PACC, D), jnp.float32),
        ],
    )


def _pool_call(h, bpad):
    return _pool_kernel()(h, bpad)


def _mm_body(h_ref, w_ref, o_ref):
    o_ref[...] = jnp.dot(h_ref[0], w_ref[0],
                         preferred_element_type=jnp.float32)


def _mm_call(h_all, w):
    return pl.pallas_call(
        _mm_body,
        grid=(T, NB),
        in_specs=[
            pl.BlockSpec((1, BN, D), lambda j, i: (j, i, 0)),
            pl.BlockSpec((1, D, D), lambda j, i: (j, 0, 0)),
        ],
        out_specs=pl.BlockSpec((BN, D), lambda j, i: (j * NB + i, 0)),
        out_shape=jax.ShapeDtypeStruct((TAB_R, D), jnp.float32),
    )(h_all, w)


def _gru_body(s_ref, inv_ref, h_ref, wi_ref, wh_ref, bi_ref, bh_ref, o_ref):
    agg = s_ref[...] * inv_ref[...]
    hb = h_ref[0]
    gi = jnp.dot(agg, wi_ref[0], preferred_element_type=jnp.float32) + bi_ref[0]
    gh = jnp.dot(hb, wh_ref[0], preferred_element_type=jnp.float32) + bh_ref[0]
    r = jax.nn.sigmoid(gi[:, :D] + gh[:, :D])
    z = jax.nn.sigmoid(gi[:, D:2 * D] + gh[:, D:2 * D])
    n = jnp.tanh(gi[:, 2 * D:] + r * gh[:, 2 * D:])
    o_ref[0] = (1.0 - z) * n + z * hb


def _gru_call(sums, inv_in, h_all, wiT, whT, bi, bh):
    return pl.pallas_call(
        _gru_body,
        grid=(T, NB),
        in_specs=[
            pl.BlockSpec((BN, D), lambda j, i: (j * NB + i, 0)),
            pl.BlockSpec((BN, 1), lambda j, i: (j * NB + i, 0)),
            pl.BlockSpec((1, BN, D), lambda j, i: (j, i, 0)),
            pl.BlockSpec((1, D, 3 * D), lambda j, i: (j, 0, 0)),
            pl.BlockSpec((1, D, 3 * D), lambda j, i: (j, 0, 0)),
            pl.BlockSpec((1, 1, 3 * D), lambda j, i: (j, 0, 0)),
            pl.BlockSpec((1, 1, 3 * D), lambda j, i: (j, 0, 0)),
        ],
        out_specs=pl.BlockSpec((1, BN, D), lambda j, i: (j, i, 0)),
        out_shape=jax.ShapeDtypeStruct((T, N, D), jnp.float32),
    )(sums, inv_in, h_all, wiT, whT, bi, bh)


def _epi_body(h_ref, c_ref, o_ref):
    hb = h_ref[...]
    cc = c_ref[...]
    comb = hb[0] * cc[0] + hb[1] * cc[1] + hb[2] * cc[2]
    o_ref[...] = jnp.broadcast_to(comb[None], (T, BN, D))


def _epi_call(h_all, coef_in):
    return pl.pallas_call(
        _epi_body,
        grid=(NB,),
        in_specs=[
            pl.BlockSpec((T, BN, D), lambda i: (0, i, 0)),
            pl.BlockSpec((T, 1, D), lambda i: (0, 0, 0)),
        ],
        out_specs=pl.BlockSpec((T, BN, D), lambda i: (0, i, 0)),
        out_shape=jax.ShapeDtypeStruct((T, N, D), jnp.float32),
    )(h_all, coef_in)


def _epf_body(h_ref, c_ref, o_ref):
    hb = h_ref[...]
    cc = c_ref[...]
    o_ref[...] = hb[0] * cc[0] + hb[1] * cc[1] + hb[2] * cc[2]


def _epf_call(h_all, coef_in):
    return pl.pallas_call(
        _epf_body,
        grid=(NB,),
        in_specs=[
            pl.BlockSpec((T, BN, D), lambda i: (0, i, 0)),
            pl.BlockSpec((T, 1, D), lambda i: (0, 0, 0)),
        ],
        out_specs=pl.BlockSpec((BN, D), lambda i: (i, 0)),
        out_shape=jax.ShapeDtypeStruct((NP_PAD, D), jnp.float32),
    )(h_all, coef_in)


def _fc_body(ps_ref, pc_ref, pt_ref, w1a_ref, w1b_ref, b1_ref, w2_ref,
             b2_ref, wl_ref, bl_ref, o_ref):
    sums = (ps_ref[0] + ps_ref[1])[:G]
    cnt = (pc_ref[0] + pc_ref[1])[:G, 0:1]
    pooled = sums / jnp.maximum(cnt, 1.0)
    x = (jnp.dot(pooled, w1a_ref[...], preferred_element_type=jnp.float32)
         + jnp.dot(pt_ref[...], w1b_ref[...], preferred_element_type=jnp.float32)
         + b1_ref[...])
    x = jnp.where(x > 0, x, 0.01 * x)
    x = jnp.dot(x, w2_ref[...], preferred_element_type=jnp.float32) + b2_ref[...]
    x = jnp.where(x > 0, x, 0.01 * x)
    o_ref[...] = jnp.dot(x, wl_ref[...], preferred_element_type=jnp.float32) + bl_ref[...]


def _fc_call(ps, pc, pt, w1a, w1b, b1, w2, b2, wl, bl):
    return pl.pallas_call(
        _fc_body,
        out_shape=jax.ShapeDtypeStruct((G, 2), jnp.float32),
    )(ps, pc, pt, w1a, w1b, b1, w2, b2, wl, bl)


def kernel(x, edge_index, edge_attr, batch, problemType,
           ggc0_weight, ggc0_w_ih, ggc0_w_hh, ggc0_b_ih, ggc0_b_hh,
           ggc1_weight, ggc1_w_ih, ggc1_w_hh, ggc1_b_ih, ggc1_b_hh,
           ggc2_weight, ggc2_w_ih, ggc2_w_hh, ggc2_b_ih, ggc2_b_hh,
           fc1_W, fc1_b, fc2_W, fc2_b, fcLast_W, fcLast_b):
    f32 = jnp.float32
    src = edge_index[0].astype(jnp.int32)
    dst = edge_index[1].astype(jnp.int32)
    ea = edge_attr.astype(jnp.int32)
    onehot = ea[None, :] == jnp.arange(T, dtype=jnp.int32)[:, None]
    present = jnp.any(onehot, axis=1)
    rank = jnp.cumsum(present.astype(jnp.int32)) - 1
    num_vals = jnp.sum(present.astype(f32))
    slot = jnp.take(rank, ea)
    g = slot * N + src
    sdx = slot * N + dst
    gflat = jnp.concatenate([g, jnp.zeros((E_PAD - E,), jnp.int32)])
    sflat = jnp.concatenate([sdx, jnp.full((E_PAD - E,), R3, jnp.int32)])
    p_parts = []
    for qq in range(NQ):
        lo = qq * QUAR
        sq = jnp.where((sflat >= lo) & (sflat < lo + QUAR), sflat - lo, LDUMP)
        p_parts.append(gflat | (sq << 15))
    coef = (num_vals > jnp.arange(T, dtype=f32)).astype(f32) / num_vals
    coef_in = jnp.broadcast_to(coef[:, None, None], (T, 1, D))

    cnt2 = _counts_call(p_parts)
    inv_in = 1.0 / jnp.maximum(cnt2, 1.0)

    w_stack = jnp.stack([ggc0_weight, ggc1_weight, ggc2_weight])
    wiT = jnp.stack([ggc0_w_ih.T, ggc1_w_ih.T, ggc2_w_ih.T])
    whT = jnp.stack([ggc0_w_hh.T, ggc1_w_hh.T, ggc2_w_hh.T])
    bi = jnp.stack([ggc0_b_ih, ggc1_b_ih, ggc2_b_ih])[:, None, :]
    bh = jnp.stack([ggc0_b_hh, ggc1_b_hh, ggc2_b_hh])[:, None, :]

    h_all = jnp.broadcast_to(x[None], (T, N, D))
    for p in range(PASSES):
        for l in range(2):
            tbl = _mm_call(h_all, w_stack[:, l])
            sums = _segsum_call(tbl, p_parts)
            h_all = _gru_call(sums, inv_in, h_all, wiT, whT, bi, bh)
        if p + 1 < PASSES:
            h_all = _epi_call(h_all, coef_in)
    h_fin = _epf_call(h_all, coef_in)

    bpad = jnp.concatenate(
        [batch.astype(jnp.int32), jnp.full((NP_PAD - N,), PDUMP, jnp.int32)])
    ps, pc = _pool_call(h_fin, bpad)

    return _fc_call(
        ps, pc, problemType,
        fc1_W[:, :D].T, fc1_W[:, D:].T, fc1_b[None],
        fc2_W.T, fc2_b[None], fcLast_W.T, fcLast_b[None],
    )
